# baseline (speedup 1.0000x reference)
import jax
import jax.numpy as jnp
from jax import lax
from jax.experimental import pallas as pl
from jax.experimental.pallas import tpu as pltpu
from jax.experimental.pallas import tpu_sc as plsc

N_NODES = 10000
CH = 128
N_EDGES = 320000

NC = 2
NS = 16
NW = NC * NS

N_PAD = 10112
ROWS_PER_TILE = N_PAD // NS

K = 88
EA = N_EDGES // NW
C0 = 128
C1 = 104
NCHUNK_PAD = NS * (C0 + C1)
E_PAD = NCHUNK_PAD * K


def _mesh():
    return plsc.VectorSubcoreMesh(
        core_axis_name="c", subcore_axis_name="s", num_cores=NC, num_subcores=NS
    )


def _deg_body(dst_hbm, out_hbm, dst_v, hist_v):
    cid = lax.axis_index("c")
    sid = lax.axis_index("s")
    wid = cid * NS + sid
    pltpu.sync_copy(dst_hbm.at[pl.ds(wid * EA, EA)], dst_v)
    zeros = jnp.zeros((16,), jnp.float32)

    def zbody(i, c):
        hist_v[pl.ds(i * 16, 16)] = zeros
        return c

    lax.fori_loop(0, N_PAD // 16, zbody, 0)
    ones = jnp.ones((16,), jnp.float32)

    def body(i, c):
        idx = dst_v[pl.ds(i * 16, 16)]
        plsc.addupdate_scatter(hist_v, [idx], ones)
        return c

    lax.fori_loop(0, EA // 16, body, 0)
    pltpu.sync_copy(hist_v, out_hbm.at[wid])


@jax.jit
def _deg_call(dst):
    return pl.kernel(
        _deg_body,
        out_type=jax.ShapeDtypeStruct((NW, N_PAD), jnp.float32),
        mesh=_mesh(),
        scratch_types=[
            pltpu.VMEM((EA,), jnp.int32),
            pltpu.VMEM((N_PAD,), jnp.float32),
        ],
        compiler_params=pltpu.CompilerParams(needs_layout_passes=False),
    )(dst)


def _lin_body(x_ref, w_ref, degp_ref, y_ref):
    deg = jnp.sum(degp_ref[...], axis=0) + 1.0
    dinv = lax.rsqrt(deg)
    xw = jnp.dot(x_ref[...], w_ref[...], preferred_element_type=jnp.float32)
    y_ref[...] = xw * dinv[:, None]


BN = N_PAD


@jax.jit
def _lin_call(x_pad, W, degp):
    return pl.pallas_call(
        _lin_body,
        grid=(N_PAD // BN,),
        in_specs=[
            pl.BlockSpec((BN, CH), lambda i: (i, 0)),
            pl.BlockSpec((CH, CH), lambda i: (0, 0)),
            pl.BlockSpec((NW, BN), lambda i: (0, i)),
        ],
        out_specs=pl.BlockSpec((BN, CH), lambda i: (i, 0)),
        out_shape=jax.ShapeDtypeStruct((N_PAD, CH), jnp.float32),
    )(x_pad, W, degp)


def _agg_body(
    y_hbm, sd_hbm, out_hbm,
    ibuf, rows0, rows1, rows2, rows3, acc_sh,
    isem0, isem1, isem2, isem3, isem4, isem5, isem6, isem7,
    gsem0, gsem1, gsem2, gsem3,
):
    cid = lax.axis_index("c")
    sid = lax.axis_index("s")
    my_cpw = jnp.where(cid == 0, C0, C1)
    base = jnp.where(cid == 0, sid * C0, NS * C0 + sid * C1)
    zeros = jnp.zeros((16,), jnp.float32)
    ncol = CH // 16

    def zb(i, c):
        rows0[i // ncol, pl.ds((i % ncol) * 16, 16)] = zeros
        return c

    lax.fori_loop(0, K * ncol, zb, 0)

    tbase = sid * ROWS_PER_TILE

    def zslab(j, c):
        pltpu.sync_copy(rows0, acc_sh.at[pl.ds(tbase + j * K, K)])
        return c

    lax.fori_loop(0, ROWS_PER_TILE // K, zslab, 0)
    ztail = ROWS_PER_TILE - (ROWS_PER_TILE // K) * K
    pltpu.sync_copy(
        rows0.at[pl.ds(0, ztail)],
        acc_sh.at[pl.ds(tbase + (ROWS_PER_TILE // K) * K, ztail)],
    )
    plsc.subcore_barrier()

    rows = (rows0, rows1, rows2, rows3)
    isems = (isem0, isem1, isem2, isem3, isem4, isem5, isem6, isem7)
    gsems = (gsem0, gsem1, gsem2, gsem3)

    for u in range(8):
        pltpu.async_copy(sd_hbm.at[base + u], ibuf.at[u], isems[u])
    for u in range(4):
        pltpu.make_async_copy(sd_hbm.at[base + u], ibuf.at[u], isems[u]).wait()
        pltpu.async_copy(y_hbm.at[ibuf.at[u, 0]], rows[u], gsems[u])

    def body(i, c):
        for u in range(8):
            q = i * 8 + u
            rb = u % 4
            s3 = (u + 4) % 8
            pltpu.make_async_copy(
                y_hbm.at[ibuf.at[u, 0]], rows[rb], gsems[rb]
            ).wait()
            pltpu.sync_copy(rows[rb], acc_sh.at[ibuf.at[u, 1]], add=True)

            @pl.when(q + 4 < my_cpw)
            def _():
                pltpu.make_async_copy(
                    sd_hbm.at[base + q + 4], ibuf.at[s3], isems[s3]
                ).wait()
                pltpu.async_copy(y_hbm.at[ibuf.at[s3, 0]], rows[rb], gsems[rb])

            @pl.when(q + 8 < my_cpw)
            def _():
                pltpu.async_copy(sd_hbm.at[base + q + 8], ibuf.at[u], isems[u])

        return c

    lax.fori_loop(0, my_cpw // 8, body, 0)

    plsc.subcore_barrier()
    sl = pl.ds(tbase, ROWS_PER_TILE)
    pltpu.sync_copy(acc_sh.at[sl], out_hbm.at[cid].at[sl])


@jax.jit
def _agg_call(y, sd):
    return pl.kernel(
        _agg_body,
        out_type=jax.ShapeDtypeStruct((NC, N_PAD, CH), jnp.float32),
        mesh=_mesh(),
        scratch_types=[
            pltpu.VMEM((8, 2, K), jnp.int32),
            pltpu.VMEM((K, CH), jnp.float32),
            pltpu.VMEM((K, CH), jnp.float32),
            pltpu.VMEM((K, CH), jnp.float32),
            pltpu.VMEM((K, CH), jnp.float32),
            pltpu.VMEM_SHARED((N_PAD, CH), jnp.float32),
            pltpu.SemaphoreType.DMA,
            pltpu.SemaphoreType.DMA,
            pltpu.SemaphoreType.DMA,
            pltpu.SemaphoreType.DMA,
            pltpu.SemaphoreType.DMA,
            pltpu.SemaphoreType.DMA,
            pltpu.SemaphoreType.DMA,
            pltpu.SemaphoreType.DMA,
            pltpu.SemaphoreType.DMA,
            pltpu.SemaphoreType.DMA,
            pltpu.SemaphoreType.DMA,
            pltpu.SemaphoreType.DMA,
        ],
    )(y, sd)


def _fin_body(accp_ref, y_ref, degp_ref, b_ref, o_ref):
    deg = jnp.sum(degp_ref[...], axis=0) + 1.0
    dinv = lax.rsqrt(deg)
    s = accp_ref[0] + accp_ref[1] + y_ref[...]
    o_ref[...] = jnp.maximum(s * dinv[:, None] + b_ref[...], 0.0)


@jax.jit
def _fin_call(accp, y, degp, b2):
    return pl.pallas_call(
        _fin_body,
        grid=(N_PAD // BN,),
        in_specs=[
            pl.BlockSpec((NC, BN, CH), lambda i: (0, i, 0)),
            pl.BlockSpec((BN, CH), lambda i: (i, 0)),
            pl.BlockSpec((NW, BN), lambda i: (0, i)),
            pl.BlockSpec((1, CH), lambda i: (0, 0)),
        ],
        out_specs=pl.BlockSpec((BN, CH), lambda i: (i, 0)),
        out_shape=jax.ShapeDtypeStruct((N_PAD, CH), jnp.float32),
    )(accp, y, degp, b2)


def kernel(x, edge_index, W, b):
    src = edge_index[0].astype(jnp.int32)
    dst = edge_index[1].astype(jnp.int32)
    x_pad = jnp.zeros((N_PAD, CH), jnp.float32).at[:N_NODES].set(x)
    srcp = jnp.full((E_PAD,), N_NODES, jnp.int32).at[:N_EDGES].set(src)
    dstp = jnp.full((E_PAD,), N_NODES, jnp.int32).at[:N_EDGES].set(dst)
    sd = jnp.stack(
        [srcp.reshape(NCHUNK_PAD, K), dstp.reshape(NCHUNK_PAD, K)], axis=1
    )

    degp = _deg_call(dst)
    y = _lin_call(x_pad, W, degp)
    accp = _agg_call(y, sd)
    out = _fin_call(accp, y, degp, b.reshape(1, CH))
    return out[:N_NODES]


# speedup vs baseline: 1.6688x; 1.6688x over previous
import jax
import jax.numpy as jnp
from jax import lax
from jax.experimental import pallas as pl
from jax.experimental.pallas import tpu as pltpu
from jax.experimental.pallas import tpu_sc as plsc

N_NODES = 10000
CH = 128
N_EDGES = 320000

NC = 2
NS = 16
NW = NC * NS

N_PAD = 10112
ROWS_PER_TILE = N_PAD // NS

K = 120
EA = N_EDGES // NW
C0 = 96
C1 = 72
NCHUNK_PAD = NS * (C0 + C1)
E_PAD = NCHUNK_PAD * K


def _mesh():
    return plsc.VectorSubcoreMesh(
        core_axis_name="c", subcore_axis_name="s", num_cores=NC, num_subcores=NS
    )


def _deg_body(dst_hbm, out_hbm, dst_v, hist_v):
    cid = lax.axis_index("c")
    sid = lax.axis_index("s")
    wid = cid * NS + sid
    pltpu.sync_copy(dst_hbm.at[pl.ds(wid * EA, EA)], dst_v)
    zeros = jnp.zeros((16,), jnp.float32)

    def zbody(i, c):
        hist_v[pl.ds(i * 16, 16)] = zeros
        return c

    lax.fori_loop(0, N_PAD // 16, zbody, 0)
    ones = jnp.ones((16,), jnp.float32)

    def body(i, c):
        idx = dst_v[pl.ds(i * 16, 16)]
        plsc.addupdate_scatter(hist_v, [idx], ones)
        return c

    lax.fori_loop(0, EA // 16, body, 0)
    pltpu.sync_copy(hist_v, out_hbm.at[wid])


@jax.jit
def _deg_call(dst):
    return pl.kernel(
        _deg_body,
        out_type=jax.ShapeDtypeStruct((NW, N_PAD), jnp.float32),
        mesh=_mesh(),
        scratch_types=[
            pltpu.VMEM((EA,), jnp.int32),
            pltpu.VMEM((N_PAD,), jnp.float32),
        ],
        compiler_params=pltpu.CompilerParams(needs_layout_passes=False),
    )(dst)


def _lin_body(x_ref, w_ref, degp_ref, y_ref):
    deg = jnp.sum(degp_ref[...], axis=0) + 1.0
    dinv = lax.rsqrt(deg)
    xw = jnp.dot(x_ref[...], w_ref[...], preferred_element_type=jnp.float32)
    y_ref[...] = xw * dinv[:, None]


BN = N_PAD


@jax.jit
def _lin_call(x_pad, W, degp):
    return pl.pallas_call(
        _lin_body,
        grid=(N_PAD // BN,),
        in_specs=[
            pl.BlockSpec((BN, CH), lambda i: (i, 0)),
            pl.BlockSpec((CH, CH), lambda i: (0, 0)),
            pl.BlockSpec((NW, BN), lambda i: (0, i)),
        ],
        out_specs=pl.BlockSpec((BN, CH), lambda i: (i, 0)),
        out_shape=jax.ShapeDtypeStruct((N_PAD, CH), jnp.float32),
    )(x_pad, W, degp)


def _agg_body(
    y_hbm, sd_hbm, out_hbm,
    ibuf, rows0, rows1, rows2, acc_sh,
    isem0, isem1, isem2, isem3, isem4, isem5, gsem0, gsem1, gsem2,
):
    cid = lax.axis_index("c")
    sid = lax.axis_index("s")
    my_cpw = jnp.where(cid == 0, C0, C1)
    base = jnp.where(cid == 0, sid * C0, NS * C0 + sid * C1)
    zeros = jnp.zeros((16,), jnp.float32)
    ncol = CH // 16

    def zb(i, c):
        rows0[i // ncol, pl.ds((i % ncol) * 16, 16)] = zeros
        return c

    lax.fori_loop(0, K * ncol, zb, 0)

    tbase = sid * ROWS_PER_TILE

    def zslab(j, c):
        pltpu.sync_copy(rows0, acc_sh.at[pl.ds(tbase + j * K, K)])
        return c

    lax.fori_loop(0, ROWS_PER_TILE // K, zslab, 0)
    ztail = ROWS_PER_TILE - (ROWS_PER_TILE // K) * K
    pltpu.sync_copy(
        rows0.at[pl.ds(0, ztail)],
        acc_sh.at[pl.ds(tbase + (ROWS_PER_TILE // K) * K, ztail)],
    )
    plsc.subcore_barrier()

    rows = (rows0, rows1, rows2)
    isems = (isem0, isem1, isem2, isem3, isem4, isem5)
    gsems = (gsem0, gsem1, gsem2)

    for u in range(6):
        pltpu.async_copy(sd_hbm.at[base + u], ibuf.at[u], isems[u])
    for u in range(3):
        pltpu.make_async_copy(sd_hbm.at[base + u], ibuf.at[u], isems[u]).wait()
        pltpu.async_copy(y_hbm.at[ibuf.at[u, 0]], rows[u], gsems[u])

    def body(i, c):
        for u in range(6):
            q = i * 6 + u
            rb = u % 3
            s3 = (u + 3) % 6
            pltpu.make_async_copy(
                y_hbm.at[ibuf.at[u, 0]], rows[rb], gsems[rb]
            ).wait()
            pltpu.sync_copy(rows[rb], acc_sh.at[ibuf.at[u, 1]], add=True)

            @pl.when(q + 3 < my_cpw)
            def _():
                pltpu.make_async_copy(
                    sd_hbm.at[base + q + 3], ibuf.at[s3], isems[s3]
                ).wait()
                pltpu.async_copy(y_hbm.at[ibuf.at[s3, 0]], rows[rb], gsems[rb])

            @pl.when(q + 6 < my_cpw)
            def _():
                pltpu.async_copy(sd_hbm.at[base + q + 6], ibuf.at[u], isems[u])

        return c

    lax.fori_loop(0, my_cpw // 6, body, 0)

    plsc.subcore_barrier()
    sl = pl.ds(tbase, ROWS_PER_TILE)
    pltpu.sync_copy(acc_sh.at[sl], out_hbm.at[cid].at[sl])


@jax.jit
def _agg_call(y, sd):
    return pl.kernel(
        _agg_body,
        out_type=jax.ShapeDtypeStruct((NC, N_PAD, CH), jnp.float32),
        mesh=_mesh(),
        scratch_types=[
            pltpu.VMEM((6, 2, K), jnp.int32),
            pltpu.VMEM((K, CH), jnp.float32),
            pltpu.VMEM((K, CH), jnp.float32),
            pltpu.VMEM((K, CH), jnp.float32),
            pltpu.VMEM_SHARED((N_PAD, CH), jnp.float32),
            pltpu.SemaphoreType.DMA,
            pltpu.SemaphoreType.DMA,
            pltpu.SemaphoreType.DMA,
            pltpu.SemaphoreType.DMA,
            pltpu.SemaphoreType.DMA,
            pltpu.SemaphoreType.DMA,
            pltpu.SemaphoreType.DMA,
            pltpu.SemaphoreType.DMA,
            pltpu.SemaphoreType.DMA,
        ],
    )(y, sd)


def _fin_body(accp_ref, y_ref, degp_ref, b_ref, o_ref):
    deg = jnp.sum(degp_ref[...], axis=0) + 1.0
    dinv = lax.rsqrt(deg)
    s = accp_ref[0] + accp_ref[1] + y_ref[...]
    o_ref[...] = jnp.maximum(s * dinv[:, None] + b_ref[...], 0.0)


@jax.jit
def _fin_call(accp, y, degp, b2):
    return pl.pallas_call(
        _fin_body,
        grid=(N_PAD // BN,),
        in_specs=[
            pl.BlockSpec((NC, BN, CH), lambda i: (0, i, 0)),
            pl.BlockSpec((BN, CH), lambda i: (i, 0)),
            pl.BlockSpec((NW, BN), lambda i: (0, i)),
            pl.BlockSpec((1, CH), lambda i: (0, 0)),
        ],
        out_specs=pl.BlockSpec((BN, CH), lambda i: (i, 0)),
        out_shape=jax.ShapeDtypeStruct((N_PAD, CH), jnp.float32),
    )(accp, y, degp, b2)


def kernel(x, edge_index, W, b):
    src = edge_index[0].astype(jnp.int32)
    dst = edge_index[1].astype(jnp.int32)
    x_pad = jnp.zeros((N_PAD, CH), jnp.float32).at[:N_NODES].set(x)
    srcp = jnp.full((E_PAD,), N_NODES, jnp.int32).at[:N_EDGES].set(src)
    dstp = jnp.full((E_PAD,), N_NODES, jnp.int32).at[:N_EDGES].set(dst)
    sd = jnp.stack(
        [srcp.reshape(NCHUNK_PAD, K), dstp.reshape(NCHUNK_PAD, K)], axis=1
    )

    degp = _deg_call(dst)
    y = _lin_call(x_pad, W, degp)
    accp = _agg_call(y, sd)
    out = _fin_call(accp, y, degp, b.reshape(1, CH))
    return out[:N_NODES]
